# baseline (speedup 1.0000x reference)
import functools

import jax
import jax.numpy as jnp
from jax.experimental import pallas as pl
from jax.experimental.pallas import tpu as pltpu

BN_EPS = 1e-5
K_TAPS = 3
N_OUT = 2


def _round_up(n, m):
  return ((n + m - 1) // m) * m


def _const_spec(shape):
  return pl.BlockSpec(shape, lambda i, _nd=len(shape): (0,) * _nd)


def _smem_spec():
  return pl.BlockSpec(memory_space=pltpu.MemorySpace.SMEM)


def _fused_body(B, C, Lout, F,
                lout_ref, gamma_ref, beta_ref, m_ref, x_ref,
                w1_hbm_ref, b1_ref, w2_ref, b2_ref, w3_ref, b3_ref,
                out_ref, mean_ref, var_ref, h_ref, w1_ref, dma_sem):
  f32 = jnp.float32
  bf16 = jnp.bfloat16
  bp = x_ref.shape[0]
  fp = h_ref.shape[1]

  w1_cp = pltpu.make_async_copy(w1_hbm_ref, w1_ref, dma_sem)
  w1_cp.start()

  n = jnp.float32(B) * lout_ref[0].astype(f32)
  inv_n = 1.0 / n
  inv_nm1 = 1.0 / jnp.maximum(n - 1.0, 1.0)

  xb = x_ref[...].astype(bf16).astype(f32)
  zs = [xb[:, k:k + Lout] for k in range(K_TAPS)]

  q = {}
  for k in range(K_TAPS):
    for k2 in range(k, K_TAPS):
      q[(k, k2)] = jnp.sum(zs[k] * zs[k2])

  hs = []
  means = []
  var_us = []
  for c in range(C):
    w = [m_ref[k:k + 1, c * Lout:c * Lout + 1].astype(f32)
         for k in range(K_TAPS)]
    conv_c = w[0] * zs[0] + w[1] * zs[1] + w[2] * zs[2]
    s_c = jnp.sum(conv_c.astype(bf16).astype(f32))
    ss_c = jnp.reshape(
        w[0] * w[0] * q[(0, 0)] + w[1] * w[1] * q[(1, 1)]
        + w[2] * w[2] * q[(2, 2)]
        + 2.0 * (w[0] * w[1] * q[(0, 1)] + w[0] * w[2] * q[(0, 2)]
                 + w[1] * w[2] * q[(1, 2)]), ())
    mean_c = s_c * inv_n
    var_b = ss_c * inv_n - mean_c * mean_c
    var_u = (ss_c - s_c * mean_c) * inv_nm1
    scale_c = gamma_ref[c] * jax.lax.rsqrt(var_b + BN_EPS)
    shift_c = beta_ref[c] - mean_c * scale_c
    scale_c = scale_c.astype(bf16).astype(f32)
    shift_c = shift_c.astype(bf16).astype(f32)
    means.append(jnp.reshape(mean_c, (1, 1)))
    var_us.append(jnp.reshape(var_u, (1, 1)))

    hs.append(jnp.maximum(conv_c * scale_c + shift_c, 0.0).astype(bf16))

  hs.append(jnp.zeros((bp, fp - F), bf16))
  h_ref[...] = jnp.concatenate(hs, axis=1)

  mean_ref[...] = jnp.concatenate(means, axis=1)
  var_ref[...] = jnp.concatenate(var_us, axis=1)

  w1_cp.wait()
  n_chunks = 5 if fp % (5 * 128) == 0 else 1
  chunk = fp // n_chunks
  acc = jnp.broadcast_to(b1_ref[...], (bp, b1_ref.shape[1]))
  for j in range(n_chunks):
    acc = acc + jnp.dot(h_ref[:, j * chunk:(j + 1) * chunk],
                        w1_ref[j * chunk:(j + 1) * chunk, :],
                        preferred_element_type=f32)
  a1 = jnp.maximum(acc, 0.0)
  a2 = jnp.maximum(jnp.dot(a1, w2_ref[...].astype(f32),
                           preferred_element_type=f32) + b2_ref[...], 0.0)
  res = (jnp.dot(a2, w3_ref[...].astype(f32),
                 preferred_element_type=f32) + b3_ref[...])
  out_ref[...] = res[:B, :N_OUT]


def kernel(x, M, P, Pt, gamma, beta, w1, b1, w2, b2, w3, b3, Lout):
  del P, Pt
  B, L = x.shape
  C = gamma.shape[1]
  Lout_s = L - K_TAPS + 1
  F = C * Lout_s
  Fp, H1p = w1.shape
  H2p, OUTp = w2.shape[1], w3.shape[1]
  Bp = _round_up(max(B, 1), 8)

  f32 = jnp.float32

  xk = x.astype(f32)
  if Bp != B:
    xk = jnp.pad(xk, ((0, Bp - B), (0, 0)))

  body = functools.partial(_fused_body, B, C, Lout_s, F)
  out, mean, var = pl.pallas_call(
      body,
      grid=(1,),
      in_specs=[_smem_spec(), _smem_spec(), _smem_spec(),
                _const_spec((8, Fp)),
                _const_spec((Bp, L)),
                pl.BlockSpec(memory_space=pltpu.MemorySpace.HBM),
                _const_spec((1, H1p)),
                _const_spec((H1p, H2p)), _const_spec((1, H2p)),
                _const_spec((H2p, OUTp)), _const_spec((1, OUTp))],
      out_specs=(_const_spec((B, N_OUT)),
                 _const_spec((1, C)), _const_spec((1, C))),
      out_shape=(jax.ShapeDtypeStruct((B, N_OUT), f32),
                 jax.ShapeDtypeStruct((1, C), f32),
                 jax.ShapeDtypeStruct((1, C), f32)),
      scratch_shapes=[pltpu.VMEM((Bp, Fp), jnp.bfloat16),
                      pltpu.VMEM((Fp, H1p), jnp.bfloat16),
                      pltpu.SemaphoreType.DMA],
      compiler_params=pltpu.CompilerParams(dimension_semantics=("arbitrary",)),
  )(Lout.reshape(1), gamma.reshape(C), beta.reshape(C),
    M, xk, w1, b1, w2, b2, w3, b3)

  return out, [(mean[0], var[0])]


# speedup vs baseline: 1.0021x; 1.0021x over previous
import functools

import jax
import jax.numpy as jnp
from jax.experimental import pallas as pl
from jax.experimental.pallas import tpu as pltpu

BN_EPS = 1e-5
K_TAPS = 3
N_OUT = 2


def _round_up(n, m):
  return ((n + m - 1) // m) * m


def _const_spec(shape):
  return pl.BlockSpec(shape, lambda i, _nd=len(shape): (0,) * _nd)


def _smem_spec():
  return pl.BlockSpec(memory_space=pltpu.MemorySpace.SMEM)


def _fused_body(B, C, Lout, F,
                lout_ref, gamma_ref, beta_ref, m_ref, x_ref,
                w1_hbm_ref, b1_ref, w2_ref, b2_ref, w3_ref, b3_ref,
                out_ref, mean_ref, var_ref, h_ref, w1_ref, dma_sem):
  f32 = jnp.float32
  bf16 = jnp.bfloat16
  bp = x_ref.shape[0]
  fp = h_ref.shape[1]

  w1_cp = pltpu.make_async_copy(w1_hbm_ref, w1_ref, dma_sem)
  w1_cp.start()

  n = jnp.float32(B) * lout_ref[0].astype(f32)
  inv_n = 1.0 / n
  inv_nm1 = 1.0 / jnp.maximum(n - 1.0, 1.0)

  xb = x_ref[...].astype(bf16).astype(f32)
  zs = [xb[:, k:k + Lout] for k in range(K_TAPS)]

  q = {}
  for k in range(K_TAPS):
    for k2 in range(k, K_TAPS):
      q[(k, k2)] = jnp.sum(zs[k] * zs[k2])

  hs = []
  means = []
  var_us = []
  for c in range(C):
    w = [m_ref[k:k + 1, c * Lout:c * Lout + 1].astype(f32)
         for k in range(K_TAPS)]
    conv_c = w[0] * zs[0] + w[1] * zs[1] + w[2] * zs[2]
    s_c = jnp.sum(conv_c.astype(bf16).astype(f32))
    ss_c = jnp.reshape(
        w[0] * w[0] * q[(0, 0)] + w[1] * w[1] * q[(1, 1)]
        + w[2] * w[2] * q[(2, 2)]
        + 2.0 * (w[0] * w[1] * q[(0, 1)] + w[0] * w[2] * q[(0, 2)]
                 + w[1] * w[2] * q[(1, 2)]), ())
    mean_c = s_c * inv_n
    var_b = ss_c * inv_n - mean_c * mean_c
    var_u = (ss_c - s_c * mean_c) * inv_nm1
    scale_c = gamma_ref[c] * jax.lax.rsqrt(var_b + BN_EPS)
    shift_c = beta_ref[c] - mean_c * scale_c
    scale_c = scale_c.astype(bf16).astype(f32)
    shift_c = shift_c.astype(bf16).astype(f32)
    means.append(jnp.reshape(mean_c, (1, 1)))
    var_us.append(jnp.reshape(var_u, (1, 1)))

    hs.append(jnp.maximum(conv_c * scale_c + shift_c, 0.0))

  hs.append(jnp.zeros((bp, fp - F), f32))
  h_ref[...] = jnp.concatenate(hs, axis=1)

  mean_ref[...] = jnp.concatenate(means, axis=1)
  var_ref[...] = jnp.concatenate(var_us, axis=1)

  w1_cp.wait()
  acc = jnp.dot(h_ref[...], w1_ref[...].astype(f32),
                preferred_element_type=f32) + b1_ref[...]
  a1 = jnp.maximum(acc, 0.0)
  a2 = jnp.maximum(jnp.dot(a1, w2_ref[...].astype(f32),
                           preferred_element_type=f32) + b2_ref[...], 0.0)
  res = (jnp.dot(a2, w3_ref[...].astype(f32),
                 preferred_element_type=f32) + b3_ref[...])
  out_ref[...] = res[:B, :N_OUT]


def kernel(x, M, P, Pt, gamma, beta, w1, b1, w2, b2, w3, b3, Lout):
  del P, Pt
  B, L = x.shape
  C = gamma.shape[1]
  Lout_s = L - K_TAPS + 1
  F = C * Lout_s
  Fp, H1p = w1.shape
  H2p, OUTp = w2.shape[1], w3.shape[1]
  Bp = _round_up(max(B, 1), 8)

  f32 = jnp.float32

  xk = x.astype(f32)
  if Bp != B:
    xk = jnp.pad(xk, ((0, Bp - B), (0, 0)))

  body = functools.partial(_fused_body, B, C, Lout_s, F)
  out, mean, var = pl.pallas_call(
      body,
      grid=(1,),
      in_specs=[_smem_spec(), _smem_spec(), _smem_spec(),
                _const_spec((8, Fp)),
                _const_spec((Bp, L)),
                pl.BlockSpec(memory_space=pltpu.MemorySpace.HBM),
                _const_spec((1, H1p)),
                _const_spec((H1p, H2p)), _const_spec((1, H2p)),
                _const_spec((H2p, OUTp)), _const_spec((1, OUTp))],
      out_specs=(_const_spec((B, N_OUT)),
                 _const_spec((1, C)), _const_spec((1, C))),
      out_shape=(jax.ShapeDtypeStruct((B, N_OUT), f32),
                 jax.ShapeDtypeStruct((1, C), f32),
                 jax.ShapeDtypeStruct((1, C), f32)),
      scratch_shapes=[pltpu.VMEM((Bp, Fp), f32),
                      pltpu.VMEM((Fp, H1p), jnp.bfloat16),
                      pltpu.SemaphoreType.DMA],
      compiler_params=pltpu.CompilerParams(dimension_semantics=("arbitrary",)),
  )(Lout.reshape(1), gamma.reshape(C), beta.reshape(C),
    M, xk, w1, b1, w2, b2, w3, b3)

  return out, [(mean[0], var[0])]


# speedup vs baseline: 1.0309x; 1.0287x over previous
import functools

import jax
import jax.numpy as jnp
from jax.experimental import pallas as pl
from jax.experimental.pallas import tpu as pltpu

BN_EPS = 1e-5
K_TAPS = 3
N_OUT = 2


def _round_up(n, m):
  return ((n + m - 1) // m) * m


def _const_spec(shape):
  return pl.BlockSpec(shape, lambda i, _nd=len(shape): (0,) * _nd)


def _smem_spec():
  return pl.BlockSpec(memory_space=pltpu.MemorySpace.SMEM)


def _fused_body(B, C, Lout, F,
                lout_ref, gamma_ref, beta_ref, m_ref, x_ref,
                w1_hbm_ref, b1_ref, w2_ref, b2_ref, w3_ref, b3_ref,
                out_ref, mean_ref, var_ref, h_ref, w1_ref, dma_sem):
  f32 = jnp.float32
  bf16 = jnp.bfloat16
  bp = x_ref.shape[0]
  fp = h_ref.shape[1]

  w1_cp = pltpu.make_async_copy(w1_hbm_ref, w1_ref, dma_sem)
  w1_cp.start()

  n = jnp.float32(B) * lout_ref[0].astype(f32)
  inv_n = 1.0 / n
  inv_nm1 = 1.0 / jnp.maximum(n - 1.0, 1.0)

  xb = x_ref[...].astype(bf16).astype(f32)
  zs = [xb[:, k:k + Lout] for k in range(K_TAPS)]

  hs = []
  means = []
  var_us = []
  for c in range(C):
    w = [m_ref[k:k + 1, c * Lout:c * Lout + 1].astype(f32)
         for k in range(K_TAPS)]
    conv_c = w[0] * zs[0] + w[1] * zs[1] + w[2] * zs[2]
    s_c = jnp.sum(conv_c.astype(bf16).astype(f32))
    ss_c = jnp.sum((conv_c * conv_c).astype(bf16).astype(f32))
    mean_c = s_c * inv_n
    var_b = ss_c * inv_n - mean_c * mean_c
    var_u = (ss_c - s_c * mean_c) * inv_nm1
    scale_c = gamma_ref[c] * jax.lax.rsqrt(var_b + BN_EPS)
    shift_c = beta_ref[c] - mean_c * scale_c
    scale_c = scale_c.astype(bf16).astype(f32)
    shift_c = shift_c.astype(bf16).astype(f32)
    means.append(jnp.reshape(mean_c, (1, 1)))
    var_us.append(jnp.reshape(var_u, (1, 1)))

    hs.append(jnp.maximum(conv_c * scale_c + shift_c, 0.0))

  hs.append(jnp.zeros((bp, fp - F), f32))
  h_ref[...] = jnp.concatenate(hs, axis=1)

  mean_ref[...] = jnp.concatenate(means, axis=1)
  var_ref[...] = jnp.concatenate(var_us, axis=1)

  w1_cp.wait()
  acc = jnp.dot(h_ref[...], w1_ref[...].astype(f32),
                preferred_element_type=f32) + b1_ref[...]
  a1 = jnp.maximum(acc, 0.0)
  a2 = jnp.maximum(jnp.dot(a1, w2_ref[...].astype(f32),
                           preferred_element_type=f32) + b2_ref[...], 0.0)
  res = (jnp.dot(a2, w3_ref[...].astype(f32),
                 preferred_element_type=f32) + b3_ref[...])
  out_ref[...] = res[:B, :N_OUT]


def kernel(x, M, P, Pt, gamma, beta, w1, b1, w2, b2, w3, b3, Lout):
  del P, Pt
  B, L = x.shape
  C = gamma.shape[1]
  Lout_s = L - K_TAPS + 1
  F = C * Lout_s
  Fp, H1p = w1.shape
  H2p, OUTp = w2.shape[1], w3.shape[1]
  Bp = _round_up(max(B, 1), 8)

  f32 = jnp.float32

  xk = x.astype(f32)
  if Bp != B:
    xk = jnp.pad(xk, ((0, Bp - B), (0, 0)))

  body = functools.partial(_fused_body, B, C, Lout_s, F)
  out, mean, var = pl.pallas_call(
      body,
      grid=(1,),
      in_specs=[_smem_spec(), _smem_spec(), _smem_spec(),
                _const_spec((8, Fp)),
                _const_spec((Bp, L)),
                pl.BlockSpec(memory_space=pltpu.MemorySpace.HBM),
                _const_spec((1, H1p)),
                _const_spec((H1p, H2p)), _const_spec((1, H2p)),
                _const_spec((H2p, OUTp)), _const_spec((1, OUTp))],
      out_specs=(_const_spec((B, N_OUT)),
                 _const_spec((1, C)), _const_spec((1, C))),
      out_shape=(jax.ShapeDtypeStruct((B, N_OUT), f32),
                 jax.ShapeDtypeStruct((1, C), f32),
                 jax.ShapeDtypeStruct((1, C), f32)),
      scratch_shapes=[pltpu.VMEM((Bp, Fp), f32),
                      pltpu.VMEM((Fp, H1p), jnp.bfloat16),
                      pltpu.SemaphoreType.DMA],
      compiler_params=pltpu.CompilerParams(dimension_semantics=("arbitrary",)),
  )(Lout.reshape(1), gamma.reshape(C), beta.reshape(C),
    M, xk, w1, b1, w2, b2, w3, b3)

  return out, [(mean[0], var[0])]
